# Initial kernel scaffold; baseline (speedup 1.0000x reference)
import jax
import jax.numpy as jnp
from jax.experimental import pallas as pl


def kernel(x, pos_embed, gamma, beta, batch_size):
    raise NotImplementedError("write your pallas kernel here")


# speedup vs baseline: 5.5415x; 5.5415x over previous
import jax
import jax.numpy as jnp
from jax.experimental import pallas as pl


def _ln_body(x_ref, pe_ref, g_ref, b_ref, o_ref):
    e = x_ref[...] + pe_ref[...]
    m = jnp.mean(e, axis=-1, keepdims=True)
    c = e - m
    v = jnp.mean(c * c, axis=-1, keepdims=True)
    inv = jax.lax.rsqrt(v + 1e-5)
    o_ref[...] = c * inv * g_ref[...] + b_ref[...]


def kernel(x, pos_embed, gamma, beta, batch_size):
    del batch_size
    b, nf, s, d = x.shape
    rows = b * nf
    xr = x.reshape(rows, s, d)
    g = gamma.reshape(1, 1, d)
    bt = beta.reshape(1, 1, d)

    G = 8
    grid = (rows // G,)

    out = pl.pallas_call(
        _ln_body,
        grid=grid,
        in_specs=[
            pl.BlockSpec((G, s, d), lambda i: (i, 0, 0)),
            pl.BlockSpec((s, d), lambda i: (0, 0)),
            pl.BlockSpec((1, 1, d), lambda i: (0, 0, 0)),
            pl.BlockSpec((1, 1, d), lambda i: (0, 0, 0)),
        ],
        out_specs=pl.BlockSpec((G, s, d), lambda i: (i, 0, 0)),
        out_shape=jax.ShapeDtypeStruct((rows, s, d), x.dtype),
    )(xr, pos_embed, g, bt)
    return out.reshape(b, nf, s, d)


# speedup vs baseline: 5.5466x; 1.0009x over previous
import jax
import jax.numpy as jnp
from jax.experimental import pallas as pl
from jax.experimental.pallas import tpu as pltpu


def _ln_body(x_ref, pe_ref, g_ref, b_ref, o_ref):
    e = x_ref[...] + pe_ref[...]
    m = jnp.mean(e, axis=-1, keepdims=True)
    c = e - m
    v = jnp.mean(c * c, axis=-1, keepdims=True)
    inv = jax.lax.rsqrt(v + 1e-5)
    o_ref[...] = c * inv * g_ref[...] + b_ref[...]


def kernel(x, pos_embed, gamma, beta, batch_size):
    del batch_size
    b, nf, s, d = x.shape
    rows = b * nf
    xr = x.reshape(rows, s, d)
    g = gamma.reshape(1, 1, d)
    bt = beta.reshape(1, 1, d)

    G = 8
    grid = (rows // G,)

    out = pl.pallas_call(
        _ln_body,
        grid=grid,
        in_specs=[
            pl.BlockSpec((G, s, d), lambda i: (i, 0, 0)),
            pl.BlockSpec((s, d), lambda i: (0, 0)),
            pl.BlockSpec((1, 1, d), lambda i: (0, 0, 0)),
            pl.BlockSpec((1, 1, d), lambda i: (0, 0, 0)),
        ],
        out_specs=pl.BlockSpec((G, s, d), lambda i: (i, 0, 0)),
        out_shape=jax.ShapeDtypeStruct((rows, s, d), x.dtype),
        compiler_params=pltpu.CompilerParams(
            dimension_semantics=("parallel",),
        ),
    )(xr, pos_embed, g, bt)
    return out.reshape(b, nf, s, d)
